# baseline (speedup 1.0000x reference)
import functools

import jax
import jax.numpy as jnp
from jax import lax
from jax.experimental import pallas as pl
from jax.experimental.pallas import tpu as pltpu
from jax.experimental.pallas import tpu_sc as plsc

_NA = 3
_EMB = 64
_NR = 100
_CP = 128
_TW = 3 * _CP
_BT = 256
_BP = 4096
_CHUNKS = 2


def _sc_gather_rows(table, idx, n_pad):
  info = plsc.get_sparse_core_info()
  nw = info.num_cores * info.num_subcores
  chunk = n_pad // (nw * _CHUNKS)
  mesh = plsc.VectorSubcoreMesh(core_axis_name="c", subcore_axis_name="s")

  @functools.partial(
      pl.kernel,
      mesh=mesh,
      out_type=jax.ShapeDtypeStruct((n_pad, _TW), jnp.float32),
      scratch_types=[
          pltpu.VMEM((chunk,), jnp.int32),
          pltpu.VMEM((chunk, _TW), jnp.float32),
          pltpu.SemaphoreType.DMA,
      ],
  )
  def gather(table_hbm, idx_hbm, out_hbm, idx_v, rows_v, sem):
    wid = lax.axis_index("s") * info.num_cores + lax.axis_index("c")
    for c in range(_CHUNKS):
      base = (wid * _CHUNKS + c) * chunk
      pltpu.sync_copy(idx_hbm.at[pl.ds(base, chunk)], idx_v)
      pltpu.async_copy(table_hbm.at[idx_v], rows_v, sem).wait()
      pltpu.sync_copy(rows_v, out_hbm.at[pl.ds(base, chunk)])

  return gather(table, idx)


def _rgcn_tc_body(anch_ref, m_ref,
                  c00, c01, c02, c10, c11, c12, c20, c21, c22,
                  bf0_ref, bf1_ref, bf2_ref,
                  r0_ref, r1_ref, r2_ref,
                  b0_ref, b1_ref, b2_ref, out_ref):
  a = [jnp.transpose(anch_ref[j]) for j in range(_NA)]
  h = jnp.transpose(m_ref[...])[:_EMB]
  c_refs = ((c00, c01, c02), (c10, c11, c12), (c20, c21, c22))
  bf_refs = (bf0_ref, bf1_ref, bf2_ref)
  r_refs = (r0_ref, r1_ref, r2_ref)
  b_refs = (b0_ref, b1_ref, b2_ref)
  for l in range(3):
    v = None
    for j in range(_NA):
      cj = jnp.transpose(c_refs[l][j][...])[:_NR]
      t = cj[:, None, :] * a[j][None, :, :]
      v = t if v is None else v + t
    agg = jnp.dot(bf_refs[l][...], v.reshape(_NR * _EMB, _BT),
                  preferred_element_type=jnp.float32)
    rl = r_refs[l][...]
    bias = b_refs[l][...]
    h = agg + jnp.dot(rl, h, preferred_element_type=jnp.float32) + bias
    if l < 2:
      h = jnp.maximum(h, 0.0)
      a = [jnp.maximum(jnp.dot(rl, a[j], preferred_element_type=jnp.float32)
                       + bias, 0.0)
           for j in range(_NA)]
  out_ref[...] = h


def kernel(anchor_embeddings, var_ids, edge_index, edge_type, mode_emb,
           comp0, basis0, root0, bias0,
           comp1, basis1, root1, bias1,
           comp2, basis2, root2, bias2):
  del edge_index
  b = anchor_embeddings.shape[1]
  nm = mode_emb.shape[0]

  comp_cat = jnp.concatenate([
      jnp.pad(x, ((0, 0), (0, _CP - _NR))) for x in (comp0, comp1, comp2)
  ], axis=1)
  table = jnp.concatenate([
      jnp.pad(mode_emb, ((0, 0), (0, _TW - _EMB))),
      comp_cat,
  ], axis=0)
  tj = jnp.pad(edge_type.astype(jnp.int32).reshape(b, _NA).T,
               ((0, 0), (0, _BP - b)))
  vid = jnp.pad(var_ids[:, 0].astype(jnp.int32), (0, _BP - b))
  idx = jnp.concatenate([vid] + [tj[j] + nm for j in range(_NA)])
  n_pad = 4 * _BP
  rows = _sc_gather_rows(table, idx, n_pad)

  def cmap(l, j):
    off = (1 + j) * (_BP // _BT)
    return lambda g, _off=off, _l=l: (_off + g, _l)

  bfs = [x.transpose(2, 0, 1).reshape(_EMB, _NR * _EMB)
         for x in (basis0, basis1, basis2)]
  roots_t = [x.T for x in (root0, root1, root2)]
  biases = [x.reshape(_EMB, 1) for x in (bias0, bias1, bias2)]
  wspec = lambda shape: pl.BlockSpec(shape, lambda g: tuple(0 for _ in shape))
  cspecs = [pl.BlockSpec((_BT, _CP), cmap(l, j))
            for l in range(3) for j in range(_NA)]
  out = pl.pallas_call(
      _rgcn_tc_body,
      grid=(_BP // _BT,),
      in_specs=[
          pl.BlockSpec((_NA, _BT, _EMB), lambda g: (0, g, 0)),
          pl.BlockSpec((_BT, _CP), lambda g: (g, 0)),
          *cspecs,
          wspec((_EMB, _NR * _EMB)),
          wspec((_EMB, _NR * _EMB)),
          wspec((_EMB, _NR * _EMB)),
          wspec((_EMB, _EMB)),
          wspec((_EMB, _EMB)),
          wspec((_EMB, _EMB)),
          wspec((_EMB, 1)),
          wspec((_EMB, 1)),
          wspec((_EMB, 1)),
      ],
      out_specs=pl.BlockSpec((_EMB, _BT), lambda g: (0, g)),
      out_shape=jax.ShapeDtypeStruct((_EMB, _BP), jnp.float32),
  )(anchor_embeddings, rows, *([rows] * 9),
    bfs[0], bfs[1], bfs[2], roots_t[0], roots_t[1], roots_t[2],
    biases[0], biases[1], biases[2])
  return out[:, :b].T


# speedup vs baseline: 1.0231x; 1.0231x over previous
import functools

import jax
import jax.numpy as jnp
from jax import lax
from jax.experimental import pallas as pl
from jax.experimental.pallas import tpu as pltpu
from jax.experimental.pallas import tpu_sc as plsc

_NA = 3
_EMB = 64
_NR = 100
_CP = 128
_BT = 256
_BP = 4096
_CHUNKS = 4


def _sc_gather_rows(table, idx, n_pad):
  info = plsc.get_sparse_core_info()
  nw = info.num_cores * info.num_subcores
  per = n_pad // nw
  chunk = per // _CHUNKS
  mesh = plsc.VectorSubcoreMesh(core_axis_name="c", subcore_axis_name="s")

  @functools.partial(
      pl.kernel,
      mesh=mesh,
      out_type=jax.ShapeDtypeStruct((n_pad, _CP), jnp.float32),
      scratch_types=[
          pltpu.VMEM((per,), jnp.int32),
          pltpu.VMEM((chunk, _CP), jnp.float32),
          pltpu.VMEM((chunk, _CP), jnp.float32),
          pltpu.SemaphoreType.DMA,
          pltpu.SemaphoreType.DMA,
          pltpu.SemaphoreType.DMA,
          pltpu.SemaphoreType.DMA,
      ],
  )
  def gather(table_hbm, idx_hbm, out_hbm, idx_v, buf0, buf1,
             gsem0, gsem1, wsem0, wsem1):
    wid = lax.axis_index("s") * info.num_cores + lax.axis_index("c")
    base = wid * per
    pltpu.sync_copy(idx_hbm.at[pl.ds(base, per)], idx_v)
    bufs = (buf0, buf1)
    gsems = (gsem0, gsem1)
    wsems = (wsem0, wsem1)
    g = [None, None]
    w = [None, None]
    for c in range(2):
      g[c] = pltpu.async_copy(
          table_hbm.at[idx_v.at[pl.ds(c * chunk, chunk)]], bufs[c], gsems[c])
    for c in range(_CHUNKS):
      p = c % 2
      g[p].wait()
      w[p] = pltpu.async_copy(
          bufs[p], out_hbm.at[pl.ds(base + c * chunk, chunk)], wsems[p])
      nc = c + 2
      if nc < _CHUNKS:
        w[p].wait()
        g[p] = pltpu.async_copy(
            table_hbm.at[idx_v.at[pl.ds(nc * chunk, chunk)]],
            bufs[p], gsems[p])
    w[(_CHUNKS - 2) % 2].wait()
    w[(_CHUNKS - 1) % 2].wait()

  return gather(table, idx)


def _rgcn_tc_body(anch_ref, m_ref,
                  c00, c01, c02, c10, c11, c12, c20, c21, c22,
                  bf0_ref, bf1_ref, bf2_ref,
                  r0_ref, r1_ref, r2_ref,
                  b0_ref, b1_ref, b2_ref, out_ref):
  a = [jnp.transpose(anch_ref[j]) for j in range(_NA)]
  h = jnp.transpose(m_ref[...])[:_EMB]
  c_refs = ((c00, c01, c02), (c10, c11, c12), (c20, c21, c22))
  bf_refs = (bf0_ref, bf1_ref, bf2_ref)
  r_refs = (r0_ref, r1_ref, r2_ref)
  b_refs = (b0_ref, b1_ref, b2_ref)
  for l in range(3):
    v = None
    for j in range(_NA):
      cj = jnp.transpose(c_refs[l][j][...])[:_NR]
      t = cj[:, None, :] * a[j][None, :, :]
      v = t if v is None else v + t
    agg = jnp.dot(bf_refs[l][...], v.reshape(_NR * _EMB, _BT),
                  preferred_element_type=jnp.float32)
    rl = r_refs[l][...]
    bias = b_refs[l][...]
    h = agg + jnp.dot(rl, h, preferred_element_type=jnp.float32) + bias
    if l < 2:
      h = jnp.maximum(h, 0.0)
      a = [jnp.maximum(jnp.dot(rl, a[j], preferred_element_type=jnp.float32)
                       + bias, 0.0)
           for j in range(_NA)]
  out_ref[...] = h


def kernel(anchor_embeddings, var_ids, edge_index, edge_type, mode_emb,
           comp0, basis0, root0, bias0,
           comp1, basis1, root1, bias1,
           comp2, basis2, root2, bias2):
  del edge_index
  b = anchor_embeddings.shape[1]
  nm = mode_emb.shape[0]

  table = jnp.concatenate([
      jnp.pad(mode_emb, ((0, 0), (0, _CP - _EMB))),
      jnp.pad(comp0, ((0, 0), (0, _CP - _NR))),
      jnp.pad(comp1, ((0, 0), (0, _CP - _NR))),
      jnp.pad(comp2, ((0, 0), (0, _CP - _NR))),
  ], axis=0)
  tj = jnp.pad(edge_type.astype(jnp.int32).reshape(b, _NA).T,
               ((0, 0), (0, _BP - b)))
  vid = jnp.pad(var_ids[:, 0].astype(jnp.int32), (0, _BP - b))
  segs = [vid] + [tj[j] + nm + l * _NR
                  for l in range(3) for j in range(_NA)]
  n_pad = 10 * _BP
  idx = jnp.concatenate(segs)
  rows = _sc_gather_rows(table, idx, n_pad)

  def cmap(l, j):
    off = (1 + 3 * l + j) * (_BP // _BT)
    return lambda g, _off=off: (_off + g, 0)

  bfs = [x.transpose(2, 0, 1).reshape(_EMB, _NR * _EMB)
         for x in (basis0, basis1, basis2)]
  roots_t = [x.T for x in (root0, root1, root2)]
  biases = [x.reshape(_EMB, 1) for x in (bias0, bias1, bias2)]
  wspec = lambda shape: pl.BlockSpec(shape, lambda g: tuple(0 for _ in shape))
  cspecs = [pl.BlockSpec((_BT, _CP), cmap(l, j))
            for l in range(3) for j in range(_NA)]
  out = pl.pallas_call(
      _rgcn_tc_body,
      grid=(_BP // _BT,),
      in_specs=[
          pl.BlockSpec((_NA, _BT, _EMB), lambda g: (0, g, 0)),
          pl.BlockSpec((_BT, _CP), lambda g: (g, 0)),
          *cspecs,
          wspec((_EMB, _NR * _EMB)),
          wspec((_EMB, _NR * _EMB)),
          wspec((_EMB, _NR * _EMB)),
          wspec((_EMB, _EMB)),
          wspec((_EMB, _EMB)),
          wspec((_EMB, _EMB)),
          wspec((_EMB, 1)),
          wspec((_EMB, 1)),
          wspec((_EMB, 1)),
      ],
      out_specs=pl.BlockSpec((_EMB, _BT), lambda g: (0, g)),
      out_shape=jax.ShapeDtypeStruct((_EMB, _BP), jnp.float32),
  )(anchor_embeddings, rows, *([rows] * 9),
    bfs[0], bfs[1], bfs[2], roots_t[0], roots_t[1], roots_t[2],
    biases[0], biases[1], biases[2])
  return out[:, :b].T


# speedup vs baseline: 1.2871x; 1.2580x over previous
import functools

import jax
import jax.numpy as jnp
from jax import lax
from jax.experimental import pallas as pl
from jax.experimental.pallas import tpu as pltpu
from jax.experimental.pallas import tpu_sc as plsc

_NA = 3
_EMB = 64
_NR = 100
_CP = 128
_BT = 256
_BP = 4096


def _sc_gather_rows(table, idx, n_pad):
  info = plsc.get_sparse_core_info()
  nw = info.num_cores * info.num_subcores
  per = n_pad // nw
  mesh = plsc.VectorSubcoreMesh(core_axis_name="c", subcore_axis_name="s")

  @functools.partial(
      pl.kernel,
      mesh=mesh,
      out_type=jax.ShapeDtypeStruct((n_pad, _CP), jnp.float32),
      scratch_types=[
          pltpu.VMEM((per,), jnp.int32),
          pltpu.VMEM((per, _CP), jnp.float32),
          pltpu.SemaphoreType.DMA,
      ],
  )
  def gather(table_hbm, idx_hbm, out_hbm, idx_v, rows_v, sem):
    wid = lax.axis_index("s") * info.num_cores + lax.axis_index("c")
    base = wid * per
    pltpu.sync_copy(idx_hbm.at[pl.ds(base, per)], idx_v)
    pltpu.async_copy(table_hbm.at[idx_v], rows_v, sem).wait()
    pltpu.sync_copy(rows_v, out_hbm.at[pl.ds(base, per)])

  return gather(table, idx)


def _rgcn_tc_body(anch_ref, m_ref, tj_ref,
                  ct0_ref, ct1_ref, ct2_ref,
                  bf0_ref, bf1_ref, bf2_ref,
                  r0_ref, r1_ref, r2_ref,
                  b0_ref, b1_ref, b2_ref, out_ref):
  a = [jnp.transpose(anch_ref[j]) for j in range(_NA)]
  h = jnp.transpose(m_ref[...])[:_EMB]
  iota_r = lax.broadcasted_iota(jnp.int32, (_NR, _BT), 0)
  oh = [(tj_ref[j][None, :] == iota_r).astype(jnp.float32)
        for j in range(_NA)]
  ct_refs = (ct0_ref, ct1_ref, ct2_ref)
  bf_refs = (bf0_ref, bf1_ref, bf2_ref)
  r_refs = (r0_ref, r1_ref, r2_ref)
  b_refs = (b0_ref, b1_ref, b2_ref)
  for l in range(3):
    ct = ct_refs[l][...]
    v = None
    for j in range(_NA):
      cj = jnp.dot(ct, oh[j], preferred_element_type=jnp.float32)
      t = cj[:, None, :] * a[j][None, :, :]
      v = t if v is None else v + t
    agg = jnp.dot(bf_refs[l][...], v.reshape(_NR * _EMB, _BT),
                  preferred_element_type=jnp.float32)
    rl = r_refs[l][...]
    bias = b_refs[l][...]
    h = agg + jnp.dot(rl, h, preferred_element_type=jnp.float32) + bias
    if l < 2:
      h = jnp.maximum(h, 0.0)
      a = [jnp.maximum(jnp.dot(rl, a[j], preferred_element_type=jnp.float32)
                       + bias, 0.0)
           for j in range(_NA)]
  out_ref[...] = h


def kernel(anchor_embeddings, var_ids, edge_index, edge_type, mode_emb,
           comp0, basis0, root0, bias0,
           comp1, basis1, root1, bias1,
           comp2, basis2, root2, bias2):
  del edge_index
  b = anchor_embeddings.shape[1]

  table = jnp.pad(mode_emb, ((0, 0), (0, _CP - _EMB)))
  vid = jnp.pad(var_ids[:, 0].astype(jnp.int32), (0, _BP - b))
  m_rows = _sc_gather_rows(table, vid, _BP)

  tj = jnp.pad(edge_type.astype(jnp.int32).reshape(b, _NA).T,
               ((0, 0), (0, _BP - b)))

  cts = [x.T for x in (comp0, comp1, comp2)]
  bfs = [x.transpose(2, 0, 1).reshape(_EMB, _NR * _EMB)
         for x in (basis0, basis1, basis2)]
  roots_t = [x.T for x in (root0, root1, root2)]
  biases = [x.reshape(_EMB, 1) for x in (bias0, bias1, bias2)]
  wspec = lambda shape: pl.BlockSpec(shape, lambda g: tuple(0 for _ in shape))
  out = pl.pallas_call(
      _rgcn_tc_body,
      grid=(_BP // _BT,),
      in_specs=[
          pl.BlockSpec((_NA, _BT, _EMB), lambda g: (0, g, 0)),
          pl.BlockSpec((_BT, _CP), lambda g: (g, 0)),
          pl.BlockSpec((_NA, _BT), lambda g: (0, g)),
          wspec((_NR, _NR)),
          wspec((_NR, _NR)),
          wspec((_NR, _NR)),
          wspec((_EMB, _NR * _EMB)),
          wspec((_EMB, _NR * _EMB)),
          wspec((_EMB, _NR * _EMB)),
          wspec((_EMB, _EMB)),
          wspec((_EMB, _EMB)),
          wspec((_EMB, _EMB)),
          wspec((_EMB, 1)),
          wspec((_EMB, 1)),
          wspec((_EMB, 1)),
      ],
      out_specs=pl.BlockSpec((_EMB, _BT), lambda g: (0, g)),
      out_shape=jax.ShapeDtypeStruct((_EMB, _BP), jnp.float32),
  )(anchor_embeddings, m_rows, tj,
    cts[0], cts[1], cts[2],
    bfs[0], bfs[1], bfs[2], roots_t[0], roots_t[1], roots_t[2],
    biases[0], biases[1], biases[2])
  return out[:, :b].T
